# baseline (speedup 1.0000x reference)
import functools

import jax
import jax.numpy as jnp
from jax import lax
from jax.experimental import pallas as pl
from jax.experimental.pallas import tpu as pltpu
from jax.experimental.pallas import tpu_sc as plsc

N_NODES = 100000
NPAD = 100352
ROWS2D = NPAD // 128
N_EDGES = 6400000
BLK = 2048
NBLK = N_EDGES // BLK
NC, NS = 2, 16
NW = NC * NS
CHUNK = NPAD // NS

_LN2 = 0.6931471805599453
_C3, _C5, _C7, _C9 = 1.0 / 3.0, 0.2, 1.0 / 7.0, 1.0 / 9.0
_INV65536 = 1.0 / 65536.0


def _pack_body(e_ref, su_ref, inf_ref, out_ref):
    e1 = jnp.maximum(e_ref[...] - 1.0, 0.0)
    infective = jnp.where(e1 == 0.0, inf_ref[...], 0.0)
    susceptible = jnp.where(e1 == jnp.inf, su_ref[...], 0.0)
    sq = jnp.clip(susceptible * 65536.0, 0.0, 65535.0).astype(jnp.int32)
    iq = jnp.clip(infective * 65536.0, 0.0, 65535.0).astype(jnp.int32)
    out_ref[...] = jnp.left_shift(sq, 16) | iq


def _fin_body(rs0_ref, rs1_ref, e_ref, inc_ref, u_ref, out_ref):
    row_sum = rs0_ref[...] + rs1_ref[...]
    e1 = jnp.maximum(e_ref[...] - 1.0, 0.0)
    prob = 1.0 - jnp.exp(row_sum)
    out_ref[...] = jnp.where(u_ref[...] < prob, inc_ref[...], e1)


def _log1m(x):
    bits = plsc.bitcast(x, jnp.int32)
    ef = (jnp.right_shift(bits, 23) - 127).astype(jnp.float32)
    m = plsc.bitcast((bits & 0x007FFFFF) | 0x3F800000, jnp.float32)
    z = m - 1.0
    t = z / (z + 2.0)
    u2 = t * t
    poly = 1.0 + u2 * (_C3 + u2 * (_C5 + u2 * (_C7 + u2 * _C9)))
    return ef * _LN2 + (t + t) * poly


_SC_MESH = plsc.VectorSubcoreMesh(
    core_axis_name="c", subcore_axis_name="s", num_cores=NC, num_subcores=NS)


@functools.partial(
    pl.kernel,
    out_type=jax.ShapeDtypeStruct((NC, NPAD), jnp.float32),
    mesh=_SC_MESH,
    compiler_params=pltpu.CompilerParams(needs_layout_passes=False),
    scratch_types=[
        pltpu.VMEM((NPAD,), jnp.int32),
        [pltpu.VMEM((BLK,), jnp.int32)] * 3,
        [pltpu.VMEM((BLK,), jnp.int32)] * 3,
        [pltpu.VMEM((BLK,), jnp.float32)] * 3,
        pltpu.VMEM((2048,), jnp.float32),
        pltpu.VMEM_SHARED((NPAD,), jnp.float32),
        [pltpu.SemaphoreType.DMA] * 3,
        [pltpu.SemaphoreType.DMA] * 3,
    ],
)
def _sc_edges(packed_hbm, src_hbm, dst_hbm, out_hbm,
              table_v, srcs, dsts, valss, zbuf, rowsum_sh, sin, ssc):
    c = lax.axis_index("c")
    s = lax.axis_index("s")
    wid = s * NC + c

    pltpu.sync_copy(packed_hbm, table_v)

    def _zero(i, carry):
        zbuf[pl.ds(i * 16, 16)] = jnp.zeros((16,), jnp.float32)
        return carry
    lax.fori_loop(0, 2048 // 16, _zero, 0)
    base = s * CHUNK
    for k in range(3):
        pltpu.sync_copy(zbuf, rowsum_sh.at[pl.ds(base + k * 2048, 2048)])
    pltpu.sync_copy(zbuf.at[pl.ds(0, CHUNK - 3 * 2048)],
                    rowsum_sh.at[pl.ds(base + 3 * 2048, CHUNK - 3 * 2048)])
    plsc.subcore_barrier()

    extra = jnp.minimum(wid, 21)
    start = wid * 97 + extra
    nblocks = jnp.where(wid < 21, 98, 97)

    shift16 = jnp.full((16,), 16, jnp.int32)

    def _issue_in(u, blk):
        pltpu.async_copy(src_hbm.at[blk], srcs[u], sin[u])
        pltpu.async_copy(dst_hbm.at[blk], dsts[u], sin[u])

    def _wait_in(u):
        pltpu.make_async_copy(src_hbm.at[0], srcs[u], sin[u]).wait()
        pltpu.make_async_copy(dst_hbm.at[0], dsts[u], sin[u]).wait()

    def _issue_scatter(u):
        pltpu.async_copy(valss[u], rowsum_sh.at[srcs[u]], ssc[u], add=True)

    def _drain_scatter(u):
        pltpu.make_async_copy(valss[u], rowsum_sh.at[srcs[u]], ssc[u]).wait()

    def _compute(u):
        def _row(r, rc):
            for g in range(8):
                sl = pl.ds(r * 128 + g * 16, 16)
                ps = plsc.load_gather(table_v, [srcs[u][sl]])
                pd = plsc.load_gather(table_v, [dsts[u][sl]])
                sval = lax.shift_right_logical(ps, shift16).astype(
                    jnp.float32) * _INV65536
                ival = (pd & 0xFFFF).astype(jnp.float32) * _INV65536
                valss[u][sl] = _log1m(1.0 - sval * ival)
            return rc
        lax.fori_loop(0, 16, _row, 0)

    _issue_in(0, start)

    def _outer(o, carry):
        for u in range(3):
            b = o * 3 + u
            v = (u + 1) % 3
            bn = b + 1

            @pl.when(bn < nblocks)
            def _prefetch():
                @pl.when(bn >= 3)
                def _():
                    _drain_scatter(v)
                _issue_in(v, start + bn)

            @pl.when(b < nblocks)
            def _work():
                _wait_in(u)
                _compute(u)
                _issue_scatter(u)
        return carry
    lax.fori_loop(0, 33, _outer, 0)
    for u in range(3):
        _drain_scatter(u)

    plsc.subcore_barrier()
    pltpu.sync_copy(rowsum_sh.at[pl.ds(base, CHUNK)],
                    out_hbm.at[c, pl.ds(base, CHUNK)])


def _pad2d(x):
    return jnp.pad(x, (0, NPAD - x.shape[0])).reshape(ROWS2D, 128)


def kernel(E, susceptiveness, infectiveness, incubation, edge_index):
    src3 = edge_index[0].reshape(NBLK, BLK)
    dst3 = edge_index[1].reshape(NBLK, BLK)
    e_pad = _pad2d(E)
    inc_pad = _pad2d(incubation)

    packed2d = pl.pallas_call(
        _pack_body,
        out_shape=jax.ShapeDtypeStruct((ROWS2D, 128), jnp.int32),
    )(e_pad, _pad2d(susceptiveness), _pad2d(infectiveness))
    packed = packed2d.reshape(NPAD)

    row_sum = _sc_edges(packed, src3, dst3)
    rs = row_sum.reshape(NC, ROWS2D, 128)

    u = jax.random.uniform(jax.random.key(42), (N_NODES,), dtype=jnp.float32)
    out2d = pl.pallas_call(
        _fin_body,
        out_shape=jax.ShapeDtypeStruct((ROWS2D, 128), jnp.float32),
    )(rs[0], rs[1], e_pad, inc_pad, _pad2d(u))
    return out2d.reshape(NPAD)[:N_NODES]


# speedup vs baseline: 3.9780x; 3.9780x over previous
import functools

import jax
import jax.numpy as jnp
from jax import lax
from jax.experimental import pallas as pl
from jax.experimental.pallas import tpu as pltpu
from jax.experimental.pallas import tpu_sc as plsc

N_NODES = 100000
NPAD = 100352
ROWS2D = NPAD // 128
N_EDGES = 6400000
BLK = 2048
NBLK = N_EDGES // BLK
NC, NS = 2, 16
NW = NC * NS
CHUNK = NPAD // NS

LOGTAB = 4096


def _pack_body(e_ref, su_ref, inf_ref, out_ref):
    e1 = jnp.maximum(e_ref[...] - 1.0, 0.0)
    infective = jnp.where(e1 == 0.0, inf_ref[...], 0.0)
    susceptible = jnp.where(e1 == jnp.inf, su_ref[...], 0.0)
    sq = jnp.clip(susceptible * 65536.0, 0.0, 65535.0).astype(jnp.int32)
    iq = jnp.clip(infective * 65536.0, 0.0, 65535.0).astype(jnp.int32)
    out_ref[...] = jnp.left_shift(sq, 16) | iq


def _fin_body(rs0_ref, rs1_ref, e_ref, inc_ref, u_ref, out_ref):
    row_sum = rs0_ref[...] + rs1_ref[...]
    e1 = jnp.maximum(e_ref[...] - 1.0, 0.0)
    prob = 1.0 - jnp.exp(row_sum)
    out_ref[...] = jnp.where(u_ref[...] < prob, inc_ref[...], e1)


_SC_MESH = plsc.VectorSubcoreMesh(
    core_axis_name="c", subcore_axis_name="s", num_cores=NC, num_subcores=NS)


@functools.partial(
    pl.kernel,
    out_type=jax.ShapeDtypeStruct((NC, NPAD), jnp.float32),
    mesh=_SC_MESH,
    compiler_params=pltpu.CompilerParams(needs_layout_passes=False),
    scratch_types=[
        pltpu.VMEM((NPAD,), jnp.int32),
        pltpu.VMEM((LOGTAB,), jnp.float32),
        [pltpu.VMEM((16, 128), jnp.int32)] * 3,
        [pltpu.VMEM((16, 128), jnp.int32)] * 3,
        [pltpu.VMEM((16, 128), jnp.float32)] * 3,
        pltpu.VMEM_SHARED((NPAD,), jnp.float32),
        [pltpu.SemaphoreType.DMA] * 3,
        [pltpu.SemaphoreType.DMA] * 3,
    ],
)
def _sc_edges(packed_hbm, ltab_hbm, zeros_hbm, src_hbm, dst_hbm, out_hbm,
              table_v, ltab_v, srcs, dsts, valss, rowsum_sh, sin, ssc):
    c = lax.axis_index("c")
    s = lax.axis_index("s")
    wid = s * NC + c

    pltpu.sync_copy(packed_hbm, table_v)
    pltpu.sync_copy(ltab_hbm, ltab_v)

    base = s * CHUNK
    pltpu.sync_copy(zeros_hbm, rowsum_sh.at[pl.ds(base, CHUNK)])
    plsc.subcore_barrier()

    extra = jnp.minimum(wid, 21)
    start = wid * 97 + extra
    nblocks = jnp.where(wid < 21, 98, 97)

    shift16 = jnp.full((16,), 16, jnp.int32)
    shift20 = jnp.full((16,), 20, jnp.int32)

    def _issue_in(u, blk):
        pltpu.async_copy(src_hbm.at[blk], srcs[u], sin[u])
        pltpu.async_copy(dst_hbm.at[blk], dsts[u], sin[u])

    def _wait_in(u):
        pltpu.make_async_copy(src_hbm.at[0], srcs[u], sin[u]).wait()
        pltpu.make_async_copy(dst_hbm.at[0], dsts[u], sin[u]).wait()

    def _issue_scatter(u):
        for r in range(16):
            pltpu.async_copy(valss[u].at[r], rowsum_sh.at[srcs[u].at[r]],
                             ssc[u], add=True)

    def _drain_scatter(u):
        for r in range(16):
            pltpu.make_async_copy(valss[u].at[r],
                                  rowsum_sh.at[srcs[u].at[r]],
                                  ssc[u]).wait()

    def _compute(u):
        def _row(rr, rc):
            for h in range(4):
                r = rr * 4 + h
                for g in range(8):
                    sl = pl.ds(g * 16, 16)
                    ps = plsc.load_gather(table_v, [srcs[u][r, sl]])
                    pd = plsc.load_gather(table_v, [dsts[u][r, sl]])
                    sq = lax.shift_right_logical(ps, shift16)
                    iq = pd & 0xFFFF
                    idx = lax.shift_right_logical(sq * iq, shift20)
                    valss[u][r, sl] = plsc.load_gather(ltab_v, [idx])
            return rc
        lax.fori_loop(0, 4, _row, 0)

    _issue_in(0, start)

    def _outer(o, carry):
        for u in range(3):
            b = o * 3 + u
            v = (u + 1) % 3
            bn = b + 1

            @pl.when(bn < nblocks)
            def _prefetch():
                @pl.when(bn >= 3)
                def _():
                    _drain_scatter(v)
                _issue_in(v, start + bn)

            @pl.when(b < nblocks)
            def _work():
                _wait_in(u)
                _compute(u)
                _issue_scatter(u)
        return carry
    lax.fori_loop(0, 33, _outer, 0)
    for u in range(3):
        _drain_scatter(u)

    plsc.subcore_barrier()
    pltpu.sync_copy(rowsum_sh.at[pl.ds(base, CHUNK)],
                    out_hbm.at[c, pl.ds(base, CHUNK)])


def _pad2d(x):
    return jnp.pad(x, (0, NPAD - x.shape[0])).reshape(ROWS2D, 128)


def kernel(E, susceptiveness, infectiveness, incubation, edge_index):
    src3 = edge_index[0].reshape(NBLK, 16, 128)
    dst3 = edge_index[1].reshape(NBLK, 16, 128)
    e_pad = _pad2d(E)
    inc_pad = _pad2d(incubation)

    packed2d = pl.pallas_call(
        _pack_body,
        out_shape=jax.ShapeDtypeStruct((ROWS2D, 128), jnp.int32),
    )(e_pad, _pad2d(susceptiveness), _pad2d(infectiveness))
    packed = packed2d.reshape(NPAD)

    j = jnp.arange(LOGTAB, dtype=jnp.float32)
    ltab = jnp.where(j == 0, 0.0, jnp.log1p(-(j + 0.5) / LOGTAB))

    zeros_c = jnp.zeros((CHUNK,), jnp.float32)
    row_sum = _sc_edges(packed, ltab, zeros_c, src3, dst3)
    rs = row_sum.reshape(NC, ROWS2D, 128)

    u = jax.random.uniform(jax.random.key(42), (N_NODES,), dtype=jnp.float32)
    out2d = pl.pallas_call(
        _fin_body,
        out_shape=jax.ShapeDtypeStruct((ROWS2D, 128), jnp.float32),
    )(rs[0], rs[1], e_pad, inc_pad, _pad2d(u))
    return out2d.reshape(NPAD)[:N_NODES]
